# baseline (speedup 1.0000x reference)
import functools

import jax
import jax.numpy as jnp
from jax.experimental import pallas as pl
from jax.experimental.pallas import tpu as pltpu
from jax.experimental.pallas import tpu_sc as plsc

_WINDOW = 128


def kernel(indices, weight):
    batch, seq = indices.shape
    vocab, dim = weight.shape
    n = batch * seq

    flat_idx = indices.reshape(1, n).astype(jnp.int32)
    pair_idx = flat_idx >> 1
    w3 = weight.reshape(vocab // 2, 2 * dim)
    mesh = plsc.VectorSubcoreMesh(core_axis_name="c", subcore_axis_name="s")

    @functools.partial(
        pl.kernel,
        out_type=jax.ShapeDtypeStruct((n, 2 * dim), weight.dtype),
        mesh=mesh,
    )
    def gather_kernel(w_hbm, i_hbm, o_hbm):
        def body(i_vmem, o_vmem):
            pltpu.sync_copy(w_hbm.at[i_vmem.at[0]], o_vmem)

        pltpu.emit_pipeline(
            body,
            grid=(n // _WINDOW,),
            in_specs=[pl.BlockSpec((1, _WINDOW), index_map=lambda i: (0, i))],
            out_specs=[
                pl.BlockSpec((_WINDOW, 2 * dim), index_map=lambda i: (i, 0))
            ],
            core_axis_name=("c", "s"),
            dimension_semantics=(pltpu.PARALLEL,),
        )(i_hbm, o_hbm)

    out2 = gather_kernel(w3, pair_idx)
    out3 = out2.reshape(n, 2, dim)
    parity = (flat_idx[0] & 1).astype(jnp.int32)
    out = jnp.where(parity[:, None] == 1, out3[:, 1, :], out3[:, 0, :])
    return out.reshape(batch, seq, dim)


# speedup vs baseline: 1.4702x; 1.4702x over previous
import functools

import jax
import jax.numpy as jnp
from jax.experimental import pallas as pl
from jax.experimental.pallas import tpu as pltpu
from jax.experimental.pallas import tpu_sc as plsc

_WINDOW = 128

_SEL_BLK = 8


def _select_body(p_ref, g_ref, o_ref):
    p = p_ref[...]
    g = g_ref[...]
    lo = g[:, :, :64]
    hi = g[:, :, 64:]
    o_ref[...] = jnp.where(p[:, :, None] > 0, hi, lo)


def kernel(indices, weight):
    batch, seq = indices.shape
    vocab, dim = weight.shape
    n = batch * seq

    flat_idx = indices.reshape(1, n).astype(jnp.int32)
    pair_idx = flat_idx >> 1
    w2 = weight.reshape(vocab // 2, 2 * dim)
    mesh = plsc.VectorSubcoreMesh(core_axis_name="c", subcore_axis_name="s")

    @functools.partial(
        pl.kernel,
        out_type=jax.ShapeDtypeStruct((n, 2 * dim), weight.dtype),
        mesh=mesh,
    )
    def gather_kernel(w_hbm, i_hbm, o_hbm):
        def body(i_vmem, o_vmem):
            pltpu.sync_copy(w_hbm.at[i_vmem.at[0]], o_vmem)

        pltpu.emit_pipeline(
            body,
            grid=(n // _WINDOW,),
            in_specs=[pl.BlockSpec((1, _WINDOW), index_map=lambda i: (0, i))],
            out_specs=[
                pl.BlockSpec((_WINDOW, 2 * dim), index_map=lambda i: (i, 0))
            ],
            core_axis_name=("c", "s"),
            dimension_semantics=(pltpu.PARALLEL,),
        )(i_hbm, o_hbm)

    out2 = gather_kernel(w2, pair_idx)

    g3 = out2.reshape(n // 128, 128, 2 * dim)
    parity = (flat_idx.reshape(n // 128, 128) & 1).astype(jnp.int32)
    grid = (n // 128) // _SEL_BLK
    out = pl.pallas_call(
        _select_body,
        grid=(grid,),
        in_specs=[
            pl.BlockSpec((_SEL_BLK, 128), lambda i: (i, 0)),
            pl.BlockSpec((_SEL_BLK, 128, 2 * dim), lambda i: (i, 0, 0)),
        ],
        out_specs=pl.BlockSpec((_SEL_BLK, 128, dim), lambda i: (i, 0, 0)),
        out_shape=jax.ShapeDtypeStruct((n // 128, 128, dim), weight.dtype),
    )(parity, g3)

    return out.reshape(batch, seq, dim)


# speedup vs baseline: 2.1635x; 1.4716x over previous
import functools

import jax
import jax.numpy as jnp
from jax import lax
from jax.experimental import pallas as pl
from jax.experimental.pallas import tpu as pltpu
from jax.experimental.pallas import tpu_sc as plsc

_NUM_CORES = 2
_NUM_SUBCORES = 16
_NUM_WORKERS = _NUM_CORES * _NUM_SUBCORES
_CHUNK = 128


def kernel(indices, weight):
    batch, seq = indices.shape
    vocab, dim = weight.shape
    n = batch * seq
    per_worker = n // _NUM_WORKERS
    n_chunks = per_worker // _CHUNK

    flat_idx = indices.reshape(1, n).astype(jnp.int32)
    w_pad = jnp.pad(weight, ((0, 0), (0, 128 - dim)))
    mesh = plsc.VectorSubcoreMesh(core_axis_name="c", subcore_axis_name="s")

    @functools.partial(
        pl.kernel,
        out_type=jax.ShapeDtypeStruct((n, 128), weight.dtype),
        mesh=mesh,
        scratch_types=[
            pltpu.VMEM((per_worker,), jnp.int32),
            pltpu.VMEM((2, _CHUNK, 128), jnp.float32),
            pltpu.SemaphoreType.DMA,
            pltpu.SemaphoreType.DMA,
            pltpu.SemaphoreType.DMA,
            pltpu.SemaphoreType.DMA,
        ],
    )
    def gather_kernel(w_hbm, i_hbm, o_hbm, idx_v, g_v, gs0, gs1, ws0, ws1):
        gsem = (gs0, gs1)
        wsem = (ws0, ws1)

        wid = lax.axis_index("s") * _NUM_CORES + lax.axis_index("c")
        base = wid * per_worker
        pltpu.sync_copy(i_hbm.at[0, pl.ds(base, per_worker)], idx_v)

        def start_gather(slot, c):
            pltpu.async_copy(
                w_hbm.at[idx_v.at[pl.ds(c * _CHUNK, _CHUNK)]],
                g_v.at[slot],
                gsem[slot],
            )

        def wait_gather(slot, c):
            pltpu.make_async_copy(
                w_hbm.at[idx_v.at[pl.ds(c * _CHUNK, _CHUNK)]],
                g_v.at[slot],
                gsem[slot],
            ).wait()

        def start_write(slot, c):
            pltpu.async_copy(
                g_v.at[slot],
                o_hbm.at[pl.ds(base + c * _CHUNK, _CHUNK)],
                wsem[slot],
            )

        def wait_write(slot, c):
            pltpu.make_async_copy(
                g_v.at[slot],
                o_hbm.at[pl.ds(base + c * _CHUNK, _CHUNK)],
                wsem[slot],
            ).wait()

        start_gather(0, 0)
        start_gather(1, 1)

        @pl.loop(0, n_chunks, step=2)
        def _(c):
            for b in range(2):
                cc = c + b
                wait_gather(b, cc)
                start_write(b, cc)

                @pl.when(cc + 2 < n_chunks)
                def _():
                    wait_write(b, cc)
                    start_gather(b, cc + 2)

        wait_write(0, n_chunks - 2)
        wait_write(1, n_chunks - 1)

    out = gather_kernel(w_pad, flat_idx)
    return out[:, :dim].reshape(batch, seq, dim)
